# Initial kernel scaffold; baseline (speedup 1.0000x reference)
import jax
import jax.numpy as jnp
from jax.experimental import pallas as pl


def kernel(features, edge_index, W_self0, W_neigh0, b0, W_self1, W_neigh1, b1):
    raise NotImplementedError("write your pallas kernel here")


# speedup vs baseline: 5.6889x; 5.6889x over previous
import functools

import jax
import jax.numpy as jnp
from jax import lax
from jax.experimental import pallas as pl
from jax.experimental.pallas import tpu as pltpu
from jax.experimental.pallas import tpu_sc as plsc

N = 10000
E = 320000
D = 128
H = 128

NC = 2
NS = 16
LANES = 16
NW = NC * NS
EPW = E // NW
CH = 80
NCHK = EPW // CH
G = 5
NGR = NCHK // G
RPS = 624
TAIL = N - NS * RPS
VPG = G * CH // LANES

_mesh = plsc.VectorSubcoreMesh(
    core_axis_name="c", subcore_axis_name="s", num_cores=NC, num_subcores=NS
)


def _copy_rows(buf, dst, start, nrows):
    nb = buf.shape[0]
    for t in range(nrows // nb):
        pltpu.sync_copy(buf, dst.at[pl.ds(start + t * nb, nb)])
    rem = nrows % nb
    if rem:
        pltpu.sync_copy(
            buf.at[pl.ds(0, rem)], dst.at[pl.ds(start + (nrows // nb) * nb, rem)]
        )


def _bounce_rows(stage, src, dst, src0, dst0, nrows):
    nb = stage.shape[0]
    done = 0
    while done < nrows:
        m = min(nb, nrows - done)
        sview = stage if m == nb else stage.at[pl.ds(0, m)]
        pltpu.sync_copy(src.at[pl.ds(src0 + done, m)], sview)
        pltpu.sync_copy(sview, dst.at[pl.ds(dst0 + done, m)])
        done += m


def _zero_fill_2d(ref, nrows, ncols):
    zeros = jnp.zeros((LANES,), jnp.float32)
    cpr = ncols // LANES

    def body(k, _):
        r = k // cpr
        c = k % cpr
        ref[r, pl.ds(c * LANES, LANES)] = zeros
        return 0

    lax.fori_loop(0, nrows * cpr, body, 0)


def _zero_fill_1d(ref, n):
    zeros = jnp.zeros((LANES,), jnp.float32)

    def body(k, _):
        ref[pl.ds(k * LANES, LANES)] = zeros
        return 0

    lax.fori_loop(0, n // LANES, body, 0)


def _agg_rows_body(table_hbm, src_hbm, dst_hbm, acc_hbm,
                   src_v, dst_v, rows_v, acc_sh, sem):
    cid = lax.axis_index("c")
    sid = lax.axis_index("s")
    wid = cid * NS + sid

    row0 = pl.multiple_of(sid * RPS, 8)
    _zero_fill_2d(rows_v, CH, D)
    _copy_rows(rows_v, acc_sh, row0, RPS)

    @pl.when(sid == NS - 1)
    def _():
        _copy_rows(rows_v, acc_sh, NS * RPS, TAIL)

    plsc.subcore_barrier()

    def group(g, _):
        pltpu.sync_copy(src_hbm.at[wid, g], src_v)
        pltpu.sync_copy(dst_hbm.at[wid, g], dst_v)
        for j in range(G):
            pltpu.async_copy(table_hbm.at[src_v.at[j]], rows_v, sem).wait()
            pltpu.sync_copy(rows_v, acc_sh.at[dst_v.at[j]], add=True)
        return 0

    lax.fori_loop(0, NGR, group, 0)

    plsc.subcore_barrier()

    _bounce_rows(rows_v, acc_sh, acc_hbm.at[cid], row0, row0, RPS)

    @pl.when(sid == NS - 1)
    def _():
        _bounce_rows(rows_v, acc_sh, acc_hbm.at[cid], NS * RPS, NS * RPS, TAIL)


_agg_rows = pl.kernel(
    _agg_rows_body,
    out_type=jax.ShapeDtypeStruct((NC, N, D), jnp.float32),
    mesh=_mesh,
    scratch_types=[
        pltpu.VMEM((G, CH), jnp.int32),
        pltpu.VMEM((G, CH), jnp.int32),
        pltpu.VMEM((CH, D), jnp.float32),
        pltpu.VMEM_SHARED((N, D), jnp.float32),
        pltpu.SemaphoreType.DMA,
    ],
)


def _fill_ones_2d(ref, nrows):
    ones = jnp.ones((LANES,), jnp.float32)

    def body(r, _):
        ref[r, pl.ds(0, LANES)] = ones
        return 0

    lax.fori_loop(0, nrows, body, 0)


def _deg_body(dst_hbm, deg_hbm, dst_v, ones_v, deg_sh):
    cid = lax.axis_index("c")
    sid = lax.axis_index("s")
    wid = cid * NS + sid
    row0 = pl.multiple_of(sid * RPS, 8)

    _zero_fill_2d(ones_v, CH, D)
    _copy_rows(ones_v, deg_sh, row0, RPS)

    @pl.when(sid == NS - 1)
    def _():
        _copy_rows(ones_v, deg_sh, NS * RPS, TAIL)

    _fill_ones_2d(ones_v, CH)
    plsc.subcore_barrier()

    def group(g, _):
        pltpu.sync_copy(dst_hbm.at[wid, g], dst_v)
        for j in range(G):
            pltpu.sync_copy(ones_v, deg_sh.at[dst_v.at[j]], add=True)
        return 0

    lax.fori_loop(0, NGR, group, 0)

    plsc.subcore_barrier()
    _bounce_rows(ones_v, deg_sh, deg_hbm.at[cid], row0, row0, RPS)

    @pl.when(sid == NS - 1)
    def _():
        _bounce_rows(ones_v, deg_sh, deg_hbm.at[cid], NS * RPS, NS * RPS, TAIL)


_deg_count = pl.kernel(
    _deg_body,
    out_type=jax.ShapeDtypeStruct((NC, N, D), jnp.float32),
    mesh=_mesh,
    scratch_types=[
        pltpu.VMEM((G, CH), jnp.int32),
        pltpu.VMEM((CH, D), jnp.float32),
        pltpu.VMEM_SHARED((N, D), jnp.float32),
    ],
)


BLK = 2000


def _mid_body(x_ref, accp_ref, degp_ref, ws0_ref, wn0_ref, b0_ref,
              ws1_ref, wn1_ref, b1_ref, srep_ref, sh_ref, degc_ref):
    x = x_ref[...]
    acc = accp_ref[0] + accp_ref[1]
    deg = degp_ref[0, :, 0] + degp_ref[1, :, 0]
    degc = jnp.maximum(deg, 1.0)
    hn = acc / degc[:, None]
    h = x @ ws0_ref[...] + hn @ wn0_ref[...] + b0_ref[...]
    h = jnp.maximum(h, 0.0)
    s = h @ wn1_ref[...]
    sh = h @ ws1_ref[...] + b1_ref[...]
    srep_ref[...] = jnp.broadcast_to(s, (BLK, D))
    sh_ref[...] = sh
    degc_ref[...] = degc[:, None]


def _mid(features, acc_p, deg_p, ws0, wn0, b0, ws1, wn1, b1):
    grid = (N // BLK,)
    return pl.pallas_call(
        _mid_body,
        grid=grid,
        in_specs=[
            pl.BlockSpec((BLK, D), lambda i: (i, 0)),
            pl.BlockSpec((NC, BLK, D), lambda i: (0, i, 0)),
            pl.BlockSpec((NC, BLK, D), lambda i: (0, i, 0)),
            pl.BlockSpec((D, H), lambda i: (0, 0)),
            pl.BlockSpec((D, H), lambda i: (0, 0)),
            pl.BlockSpec((1, H), lambda i: (0, 0)),
            pl.BlockSpec((H, 1), lambda i: (0, 0)),
            pl.BlockSpec((H, 1), lambda i: (0, 0)),
            pl.BlockSpec((1, 1), lambda i: (0, 0)),
        ],
        out_specs=[
            pl.BlockSpec((BLK, D), lambda i: (i, 0)),
            pl.BlockSpec((BLK, 1), lambda i: (i, 0)),
            pl.BlockSpec((BLK, 1), lambda i: (i, 0)),
        ],
        out_shape=[
            jax.ShapeDtypeStruct((N, D), jnp.float32),
            jax.ShapeDtypeStruct((N, 1), jnp.float32),
            jax.ShapeDtypeStruct((N, 1), jnp.float32),
        ],
    )(features, acc_p, deg_p, ws0, wn0, b0, ws1, wn1, b1)


def _final_body(sh_ref, accp_ref, degc_ref, out_ref):
    agg = accp_ref[0, :, 0:1] + accp_ref[1, :, 0:1]
    out_ref[...] = sh_ref[...] + agg / degc_ref[...]


def _final(sh, acc1_p, degc):
    grid = (N // BLK,)
    return pl.pallas_call(
        _final_body,
        grid=grid,
        in_specs=[
            pl.BlockSpec((BLK, 1), lambda i: (i, 0)),
            pl.BlockSpec((NC, BLK, D), lambda i: (0, i, 0)),
            pl.BlockSpec((BLK, 1), lambda i: (i, 0)),
        ],
        out_specs=pl.BlockSpec((BLK, 1), lambda i: (i, 0)),
        out_shape=jax.ShapeDtypeStruct((N, 1), jnp.float32),
    )(sh, acc1_p, degc)


def kernel(features, edge_index, W_self0, W_neigh0, b0, W_self1, W_neigh1, b1):
    src = edge_index[0].reshape(NW, NGR, G, CH)
    dst = edge_index[1].reshape(NW, NGR, G, CH)

    acc_p = _agg_rows(features, src, dst)
    deg_p = _deg_count(dst)
    s_rep, sh, degc = _mid(
        features, acc_p, deg_p,
        W_self0, W_neigh0, b0.reshape(1, H),
        W_self1, W_neigh1, b1.reshape(1, 1),
    )
    acc1_p = _agg_rows(s_rep, src, dst)
    return _final(sh, acc1_p, degc)
